# Initial kernel scaffold; baseline (speedup 1.0000x reference)
import jax
import jax.numpy as jnp
from jax.experimental import pallas as pl


def kernel(x_order, x_device, x_type, ei_d2o, ei_t2o, ei_o2d, ei_d2d, ei_t2d, Wpo, bpo, Wpd, bpd, Wpt, bpt, Wuo, buo, Wud, bud, g_o, b_o, g_d, b_d):
    raise NotImplementedError("write your pallas kernel here")


# speedup vs baseline: 6.0012x; 6.0012x over previous
import jax
import jax.numpy as jnp
from jax import lax
from jax.experimental import pallas as pl
from jax.experimental.pallas import tpu as pltpu
from jax.experimental.pallas import tpu_sc as plsc

H = 48
NC, NS, L = 2, 16, 16
N_ORDER = 100000
N_DEVICE = 100000
N_TYPE = 1000
E_BIG = 1600000
E_SMALL = 100000
RB = 2000

N_PAD_BIG = 100352
N_PAD_SMALL = 1024


def _elu(x):
    return jnp.where(x > 0, x, jnp.exp(x) - 1.0)


GRP = 128
CHUNK = 640
KMAX = (CHUNK + 127) // 128


def _rel_cfg(e_total, slab, passes):
    per_tile = ((e_total // NS) + 7) // 8 * 8
    last = e_total - per_tile * (NS - 1)
    assert 0 < last <= per_tile and per_tile % 8 == 0 and last % 8 == 0
    n_chunks = min(per_tile, last) // CHUNK
    tail_a = per_tile - n_chunks * CHUNK
    tail_b = last - n_chunks * CHUNK
    assert tail_a % 8 == 0 and tail_b % 8 == 0
    per = slab // NS
    assert per % 8 == 0
    return dict(e_total=e_total, slab=slab, passes=passes, per=per,
                per_tile=per_tile, n_chunks=n_chunks,
                tail_a=tail_a, tail_b=tail_b)


CFG_BIG = _rel_cfg(E_BIG, 25088, 2)
CFG_SMALL = _rel_cfg(E_SMALL, 512, 1)


def _sc_agg_all_body(hd, ho, ht, e_d2o, e_o2d, e_d2d, e_t2o, e_t2d,
                     s0, s1, s2, s3, s4, c0, c1, c2, c3, c4,
                     sidx, didx, csrc, cdst2d, rows, zbuf, zcnt, onesb,
                     ssum, scnt, sem_g, sem_s):
    cid = lax.axis_index("c")
    sid = lax.axis_index("s")
    nrows_buf = KMAX * GRP

    lane = lax.iota(jnp.int32, L)
    zeros16 = jnp.zeros((L,), jnp.float32)
    ones16 = jnp.ones((L,), jnp.float32)

    def _init_z(r, _):
        for j in range(H // L):
            zbuf[r, pl.ds(j * L, L)] = zeros16
        return 0
    lax.fori_loop(0, zbuf.shape[0], _init_z, 0)

    def _zero_zcnt():
        def _init_zc(i, _):
            zcnt[pl.ds(i * L, L)] = zeros16
            return 0
        lax.fori_loop(0, zcnt.shape[0] // L, _init_zc, 0)
    _zero_zcnt()
    for i in range(GRP // L):
        onesb[pl.ds(i * L, L)] = ones16

    trash_src = sid * L + lane

    def run_relation(table, ei, sum_out, cnt_out, cfg):
        e_total = cfg["e_total"]
        slab = cfg["slab"]
        per = cfg["per"]
        trash_dst = slab + lane

        def process_chunk(lo, hi, base_e, size):
            base_e = pl.multiple_of(base_e, 8)
            pltpu.sync_copy(ei.at[pl.ds(base_e, size)],
                            sidx.at[pl.ds(0, size)])
            pltpu.sync_copy(ei.at[pl.ds(e_total + base_e, size)],
                            didx.at[pl.ds(0, size)])

            nv_full = size // L
            rem = size % L

            def compact_one(v, n, lane_mask=None):
                dvec = didx[pl.ds(v * L, L)]
                svec = sidx[pl.ds(v * L, L)]
                dloc = dvec - lo
                m = (dvec >= lo) & (dvec < hi)
                if lane_mask is not None:
                    m = m & lane_mask
                cs = plsc.cumsum(m.astype(jnp.int32))
                pos = n + cs - 1
                plsc.store_scatter(cdst2d, [pos >> 7, pos & (GRP - 1)], dloc,
                                   mask=m)
                plsc.store_scatter(csrc, [pos], svec, mask=m)
                return n + lax.squeeze(lax.slice(cs, (L - 1,), (L,)), (0,))

            n = lax.fori_loop(0, nv_full, compact_one, jnp.int32(0))
            if rem:
                n = compact_one(nv_full, n, lane_mask=lane < rem)

            for j in range(GRP // L):
                padpos = n + j * L + lane
                plsc.store_scatter(cdst2d, [padpos >> 7, padpos & (GRP - 1)],
                                   trash_dst)
                plsc.store_scatter(csrc, [padpos], trash_src)

            for k in range(KMAX):
                @pl.when(k * GRP < n)
                def _(k=k):
                    pltpu.async_copy(table.at[csrc.at[pl.ds(k * GRP, GRP)]],
                                     rows.at[pl.ds(k * GRP, GRP)], sem_g)
            for k in range(KMAX):
                @pl.when(k * GRP < n)
                def _(k=k):
                    pltpu.make_async_copy(
                        table.at[csrc.at[pl.ds(k * GRP, GRP)]],
                        rows.at[pl.ds(k * GRP, GRP)], sem_g).wait()

            for k in range(KMAX):
                @pl.when(k * GRP < n)
                def _(k=k):
                    pltpu.async_copy(rows.at[pl.ds(k * GRP, GRP)],
                                     ssum.at[cdst2d.at[k]], sem_s, add=True)
                    pltpu.async_copy(onesb, scnt.at[cdst2d.at[k]], sem_s,
                                     add=True)
            for k in range(KMAX):
                @pl.when(k * GRP < n)
                def _(k=k):
                    pltpu.make_async_copy(rows.at[pl.ds(k * GRP, GRP)],
                                          ssum.at[cdst2d.at[k]], sem_s).wait()
                    pltpu.make_async_copy(onesb, scnt.at[cdst2d.at[k]],
                                          sem_s).wait()

        for p in range(cfg["passes"]):
            slab_idx = cid * cfg["passes"] + p
            lo = slab_idx * slab
            hi = lo + slab

            zb = sid * per
            off = 0
            while off < per:
                sz = min(per - off, zbuf.shape[0])
                pltpu.sync_copy(zbuf.at[pl.ds(0, sz)],
                                ssum.at[pl.ds(zb + off, sz)])
                off += sz
            pltpu.sync_copy(zcnt.at[pl.ds(0, per)], scnt.at[pl.ds(zb, per)])
            plsc.subcore_barrier()

            ebase = sid * cfg["per_tile"]
            tail_base = ebase + cfg["n_chunks"] * CHUNK

            def chunk_body(i, _):
                process_chunk(lo, hi, ebase + i * CHUNK, CHUNK)
                return 0
            lax.fori_loop(0, cfg["n_chunks"], chunk_body, 0)
            if cfg["tail_a"]:
                @pl.when(sid < NS - 1)
                def _():
                    process_chunk(lo, hi, tail_base, cfg["tail_a"])
            if cfg["tail_b"]:
                @pl.when(sid == NS - 1)
                def _():
                    process_chunk(lo, hi, tail_base, cfg["tail_b"])
            plsc.subcore_barrier()

            off = 0
            while off < per:
                sz = min(per - off, nrows_buf)
                pltpu.sync_copy(ssum.at[pl.ds(zb + off, sz)],
                                rows.at[pl.ds(0, sz)])
                pltpu.sync_copy(rows.at[pl.ds(0, sz)],
                                sum_out.at[pl.ds(lo + zb + off, sz)])
                off += sz
            pltpu.sync_copy(scnt.at[pl.ds(zb, per)], zcnt.at[pl.ds(0, per)])
            pltpu.sync_copy(zcnt.at[pl.ds(0, per)],
                            cnt_out.at[pl.ds(lo + zb, per)])
            _zero_zcnt()
            plsc.subcore_barrier()

    run_relation(hd, e_d2o, s0, c0, CFG_BIG)
    run_relation(ho, e_o2d, s1, c1, CFG_BIG)
    run_relation(hd, e_d2d, s2, c2, CFG_BIG)
    run_relation(ht, e_t2o, s3, c3, CFG_SMALL)
    run_relation(ht, e_t2d, s4, c4, CFG_SMALL)


_AGG_CACHE = {}


def _get_agg():
    if not _AGG_CACHE:
        mesh = plsc.VectorSubcoreMesh(core_axis_name="c",
                                      subcore_axis_name="s",
                                      num_cores=NC, num_subcores=NS)
        slab = CFG_BIG["slab"]
        big = jax.ShapeDtypeStruct((N_PAD_BIG, H), jnp.float32)
        bigc = jax.ShapeDtypeStruct((N_PAD_BIG,), jnp.float32)
        small = jax.ShapeDtypeStruct((N_PAD_SMALL, H), jnp.float32)
        smallc = jax.ShapeDtypeStruct((N_PAD_SMALL,), jnp.float32)
        _AGG_CACHE["k"] = pl.kernel(
            _sc_agg_all_body,
            out_type=(big, big, big, small, small,
                      bigc, bigc, bigc, smallc, smallc),
            mesh=mesh,
            compiler_params=pltpu.CompilerParams(use_tc_tiling_on_sc=False,
                                                 needs_layout_passes=False),
            scratch_types=[
                pltpu.VMEM((CHUNK,), jnp.int32),
                pltpu.VMEM((CHUNK,), jnp.int32),
                pltpu.VMEM((CHUNK + GRP,), jnp.int32),
                pltpu.VMEM((KMAX + 1, GRP), jnp.int32),
                pltpu.VMEM((KMAX * GRP, H), jnp.float32),
                pltpu.VMEM((GRP, H), jnp.float32),
                pltpu.VMEM((1568,), jnp.float32),
                pltpu.VMEM((GRP,), jnp.float32),
                pltpu.VMEM_SHARED((slab + L, H), jnp.float32),
                pltpu.VMEM_SHARED((slab + L,), jnp.float32),
                pltpu.SemaphoreType.DMA,
                pltpu.SemaphoreType.DMA,
            ],
        )
    return _AGG_CACHE["k"]


def _proj_body(xo, xd, xt, Wpo, bpo, Wpd, bpd, Wpt, bpt, ho, hd, ht):
    ho[...] = _elu(jnp.dot(xo[...], Wpo[...],
                           preferred_element_type=jnp.float32) + bpo[...])
    hd[...] = _elu(jnp.dot(xd[...], Wpd[...],
                           preferred_element_type=jnp.float32) + bpd[...])
    ht[...] = _elu(xt[...] * Wpt[...] + bpt[...])


def _project(xo, xd, xt, Wpo, bpo, Wpd, bpd, Wpt, bpt):
    grid = (N_ORDER // RB,)
    blk = lambda shape: pl.BlockSpec(shape, lambda i: (i, 0))
    cst = lambda shape: pl.BlockSpec(shape, lambda i: (0, 0))
    return pl.pallas_call(
        _proj_body,
        grid=grid,
        in_specs=[blk((RB, 5)), blk((RB, 6)), cst((N_TYPE, 1)),
                  cst((5, H)), cst((1, H)), cst((6, H)), cst((1, H)),
                  cst((1, H)), cst((1, H))],
        out_specs=[blk((RB, H)), blk((RB, H)), cst((N_TYPE, H))],
        out_shape=[jax.ShapeDtypeStruct((N_ORDER, H), jnp.float32),
                   jax.ShapeDtypeStruct((N_DEVICE, H), jnp.float32),
                   jax.ShapeDtypeStruct((N_TYPE, H), jnp.float32)],
    )(xo, xd, xt, Wpo, bpo.reshape(1, H), Wpd, bpd.reshape(1, H),
      Wpt.reshape(1, H), bpt.reshape(1, H))


def _final_body(ho, sod, cod, sot, cot,
                hd, sdo, cdo, sdd, cdd, sdt, cdt,
                Wuo_a, Wuo_b, buo, Wud_a, Wud_b, Wud_c, bud,
                g_o, b_o, g_d, b_d, ho_new, hd_new):
    pid = pl.program_id(0)
    first = jnp.where(pid == 0, 1.0, 0.0).astype(jnp.float32)

    agg_o = sod[...] / jnp.maximum(cod[...], 1.0)
    agg_o = agg_o + first * (sot[...] / jnp.maximum(cot[...], 1.0))
    u = (jnp.dot(ho[...], Wuo_a[...], preferred_element_type=jnp.float32)
         + jnp.dot(agg_o, Wuo_b[...], preferred_element_type=jnp.float32)
         + buo[...])
    a = _elu(u)
    m = jnp.mean(a, axis=-1, keepdims=True)
    v = jnp.mean((a - m) ** 2, axis=-1, keepdims=True)
    ho_new[...] = (a - m) / jnp.sqrt(v + 1e-5) * g_o[...] + b_o[...]

    agg_do = sdo[...] / jnp.maximum(cdo[...], 1.0)
    agg_dd = sdd[...] / jnp.maximum(cdd[...], 1.0)
    agg_dd = agg_dd + first * (sdt[...] / jnp.maximum(cdt[...], 1.0))
    u2 = (jnp.dot(hd[...], Wud_a[...], preferred_element_type=jnp.float32)
          + jnp.dot(agg_do, Wud_b[...], preferred_element_type=jnp.float32)
          + jnp.dot(agg_dd, Wud_c[...], preferred_element_type=jnp.float32)
          + bud[...])
    a2 = _elu(u2)
    m2 = jnp.mean(a2, axis=-1, keepdims=True)
    v2 = jnp.mean((a2 - m2) ** 2, axis=-1, keepdims=True)
    hd_new[...] = (a2 - m2) / jnp.sqrt(v2 + 1e-5) * g_d[...] + b_d[...]


def _finalize(ho, sod, cod, sot, cot, hd, sdo, cdo, sdd, cdd, sdt, cdt,
              Wuo, buo, Wud, bud, g_o, b_o, g_d, b_d):
    grid = (N_ORDER // RB,)
    blk = lambda shape: pl.BlockSpec(shape, lambda i: (i, 0))
    cst = lambda shape: pl.BlockSpec(shape, lambda i: (0, 0))
    return pl.pallas_call(
        _final_body,
        grid=grid,
        in_specs=[blk((RB, H)), blk((RB, H)), blk((RB, 1)),
                  cst((RB, H)), cst((RB, 1)),
                  blk((RB, H)), blk((RB, H)), blk((RB, 1)),
                  blk((RB, H)), blk((RB, 1)), cst((RB, H)), cst((RB, 1)),
                  cst((H, H)), cst((H, H)), cst((1, H)),
                  cst((H, H)), cst((H, H)), cst((H, H)), cst((1, H)),
                  cst((1, H)), cst((1, H)), cst((1, H)), cst((1, H))],
        out_specs=[blk((RB, H)), blk((RB, H))],
        out_shape=[jax.ShapeDtypeStruct((N_ORDER, H), jnp.float32),
                   jax.ShapeDtypeStruct((N_DEVICE, H), jnp.float32)],
    )(ho, sod, cod, sot, cot, hd, sdo, cdo, sdd, cdd, sdt, cdt,
      Wuo[:H], Wuo[H:], buo.reshape(1, H),
      Wud[:H], Wud[H:2 * H], Wud[2 * H:], bud.reshape(1, H),
      g_o.reshape(1, H), b_o.reshape(1, H),
      g_d.reshape(1, H), b_d.reshape(1, H))


def kernel(x_order, x_device, x_type, ei_d2o, ei_t2o, ei_o2d, ei_d2d, ei_t2d,
           Wpo, bpo, Wpd, bpd, Wpt, bpt, Wuo, buo, Wud, bud, g_o, b_o,
           g_d, b_d):
    ho, hd, ht = _project(x_order, x_device, x_type,
                          Wpo, bpo, Wpd, bpd, Wpt, bpt)

    (sod, sdo, sdd, sot, sdt, cod, cdo, cdd, cot, cdt) = _get_agg()(
        hd, ho, ht,
        ei_d2o.reshape(-1), ei_o2d.reshape(-1), ei_d2d.reshape(-1),
        ei_t2o.reshape(-1), ei_t2d.reshape(-1))

    pad_s = lambda s: jnp.pad(s, ((0, RB - N_PAD_SMALL), (0, 0)))
    pad_c = lambda c: jnp.pad(c, (0, RB - N_PAD_SMALL)).reshape(RB, 1)
    col = lambda c: c.reshape(-1, 1)

    return _finalize(ho, sod, col(cod), pad_s(sot), pad_c(cot),
                     hd, sdo, col(cdo), sdd, col(cdd), pad_s(sdt), pad_c(cdt),
                     Wuo, buo, Wud, bud, g_o, b_o, g_d, b_d)
